# baseline (speedup 1.0000x reference)
import jax
import jax.numpy as jnp
from jax.experimental import pallas as pl
from jax.experimental.pallas import tpu as pltpu

_LONGEST_LENGTH = 4096


def _pad_body(x_hbm, o_ref, scratch, sem):
    b = pl.program_id(0)
    t = pl.program_id(1)
    slot = jax.lax.rem(b, 2)

    @pl.when(t == 0)
    def _zero_step():
        pltpu.make_async_copy(
            x_hbm.at[pl.ds(2 * b, 2)], scratch.at[slot], sem
        ).start()
        o_ref[...] = jnp.zeros_like(o_ref)

    @pl.when(t == 1)
    def _copy_step():
        pltpu.make_async_copy(
            x_hbm.at[pl.ds(2 * b, 2)], scratch.at[slot], sem
        ).wait()
        o_ref[...] = scratch[slot]


def kernel(x, input_sizes_list=None, longest_length=None):
    B, T, D = x.shape
    L = _LONGEST_LENGTH
    out = pl.pallas_call(
        _pad_body,
        grid=(B // 2, 2),
        in_specs=[pl.BlockSpec(memory_space=pl.ANY)],
        out_specs=pl.BlockSpec((2, T, D), lambda b, t: (b, 1 - t, 0)),
        out_shape=jax.ShapeDtypeStruct((B, L, D), x.dtype),
        scratch_shapes=[
            pltpu.VMEM((2, 2, T, D), x.dtype),
            pltpu.SemaphoreType.DMA,
        ],
    )(x)
    return out


# speedup vs baseline: 1.1028x; 1.1028x over previous
import jax
import jax.numpy as jnp
from jax.experimental import pallas as pl
from jax.experimental.pallas import tpu as pltpu

_LONGEST_LENGTH = 4096


def _pad_body(x_ref, o_ref):
    T = x_ref.shape[1]
    o_ref[:, :T, :] = x_ref[...]
    o_ref[:, T:, :] = jnp.zeros_like(o_ref[:, T:, :])


def kernel(x, input_sizes_list=None, longest_length=None):
    B, T, D = x.shape
    L = _LONGEST_LENGTH
    bb = 2 if B % 2 == 0 else 1
    out = pl.pallas_call(
        _pad_body,
        grid=(B // bb,),
        in_specs=[pl.BlockSpec((bb, T, D), lambda b: (b, 0, 0))],
        out_specs=pl.BlockSpec((bb, L, D), lambda b: (b, 0, 0)),
        out_shape=jax.ShapeDtypeStruct((B, L, D), x.dtype),
        compiler_params=pltpu.CompilerParams(
            vmem_limit_bytes=120 * 1024 * 1024,
        ),
    )(x)
    return out
